# Initial kernel scaffold; baseline (speedup 1.0000x reference)
import functools

import jax
import jax.numpy as jnp
from jax import lax
from jax.experimental import pallas as pl
from jax.experimental.pallas import tpu as pltpu
from jax.experimental.pallas import tpu_sc as plsc

N = 10000
E = 320000
D = 128
NEG_SLOPE = 0.2

NC = 2
NS = 16
NW = NC * NS
PAD = 144
C = 128
NCHUNK = E // C
ROWS_PER_TILE = N // NS


def _project_body(x_ref, w_ref, a2_ref, xlin_ref, aa_ref):
    xl = jnp.dot(x_ref[...], w_ref[...], preferred_element_type=jnp.float32)
    xlin_ref[...] = xl
    aa_ref[...] = jnp.dot(xl, a2_ref[...], preferred_element_type=jnp.float32)


def _project(x, W, a2):
    return pl.pallas_call(
        _project_body,
        out_shape=(
            jax.ShapeDtypeStruct((N, D), jnp.float32),
            jax.ShapeDtypeStruct((N, D), jnp.float32),
        ),
    )(x, W, a2)


def _sc_body(a_src_hbm, a_dst_hbm, src_hbm, dst_hbm, xlin_hbm, out_hbm,
             a_src_v, a_dst_v, src_v, dst_v, rows_v, msg_v, w_v, acc_sh, sem):
    cid = lax.axis_index("c")
    sid = lax.axis_index("s")
    wid = sid * NC + cid

    pltpu.sync_copy(a_src_hbm, a_src_v)
    pltpu.sync_copy(a_dst_hbm, a_dst_v)

    zeros16 = jnp.zeros((16,), jnp.float32)

    def zero_row(r, _):
        for c16 in range(PAD // 16):
            msg_v[r, pl.ds(c16 * 16, 16)] = zeros16
        return 0

    lax.fori_loop(0, C, zero_row, 0)

    base_row = sid * ROWS_PER_TILE
    for i in range(4):
        pltpu.sync_copy(msg_v, acc_sh.at[pl.ds(base_row + i * C, C)])
    pltpu.sync_copy(msg_v.at[pl.ds(0, ROWS_PER_TILE - 4 * C)],
                    acc_sh.at[pl.ds(base_row + 4 * C, ROWS_PER_TILE - 4 * C)])
    plsc.subcore_barrier()

    lane = lax.iota(jnp.int32, 16)
    nk = (NCHUNK - wid + NW - 1) // NW

    def chunk(i, _):
        k = wid + i * NW
        base = k * C
        pltpu.sync_copy(src_hbm.at[pl.ds(base, C)], src_v)
        pltpu.sync_copy(dst_hbm.at[pl.ds(base, C)], dst_v)
        pltpu.async_copy(xlin_hbm.at[src_v], rows_v, sem).wait()

        def grp(g, _):
            sidx = src_v[pl.ds(g * 16, 16)]
            didx = dst_v[pl.ds(g * 16, 16)]
            logit = (plsc.load_gather(a_src_v, [sidx])
                     + plsc.load_gather(a_dst_v, [didx]))
            w16 = jnp.exp(jnp.where(logit >= 0, logit, NEG_SLOPE * logit))
            w_v[pl.ds(g * 16, 16)] = w16
            return 0

        lax.fori_loop(0, C // 16, grp, 0)

        def edge(e, _):
            wv = jnp.full((16,), w_v[e], jnp.float32)
            for c16 in range(D // 16):
                sl = pl.ds(c16 * 16, 16)
                msg_v[e, sl] = rows_v[e, sl] * wv
            msg_v[e, pl.ds(D, 16)] = jnp.where(lane == 0, wv, 0.0)
            return 0

        lax.fori_loop(0, C, edge, 0)

        pltpu.sync_copy(msg_v, acc_sh.at[dst_v], add=True)
        return 0

    lax.fori_loop(0, nk, chunk, 0)
    plsc.subcore_barrier()

    pltpu.sync_copy(acc_sh.at[pl.ds(base_row, ROWS_PER_TILE)],
                    out_hbm.at[cid, pl.ds(base_row, ROWS_PER_TILE)])


_sc_edges = functools.partial(
    pl.kernel,
    out_type=jax.ShapeDtypeStruct((NC, N, PAD), jnp.float32),
    mesh=plsc.VectorSubcoreMesh(core_axis_name="c", subcore_axis_name="s"),
    scratch_types=[
        pltpu.VMEM((N,), jnp.float32),
        pltpu.VMEM((N,), jnp.float32),
        pltpu.VMEM((C,), jnp.int32),
        pltpu.VMEM((C,), jnp.int32),
        pltpu.VMEM((C, D), jnp.float32),
        pltpu.VMEM((C, PAD), jnp.float32),
        pltpu.VMEM((C,), jnp.float32),
        pltpu.VMEM_SHARED((N, PAD), jnp.float32),
        pltpu.SemaphoreType.DMA,
    ],
)(_sc_body)


def _combine_body(acc_ref, xlin_ref, aa_ref, out_ref):
    acc0 = acc_ref[0]
    acc1 = acc_ref[1]
    msg = acc0[:, :D] + acc1[:, :D]
    den = acc0[:, D:D + 1] + acc1[:, D:D + 1]
    asum = aa_ref[:, 0:1] + aa_ref[:, 1:2]
    wself = jnp.exp(jnp.where(asum >= 0, asum, NEG_SLOPE * asum))
    out_ref[...] = (msg + wself * xlin_ref[...]) / (den + wself + 1e-16)


def _combine(acc, xlin, aa):
    return pl.pallas_call(
        _combine_body,
        out_shape=jax.ShapeDtypeStruct((N, D), jnp.float32),
    )(acc, xlin, aa)


def kernel(x, edge_index, W, att_src, att_dst):
    a2 = jnp.zeros((D, D), jnp.float32)
    a2 = a2.at[:, 0].set(att_src.reshape(-1)).at[:, 1].set(att_dst.reshape(-1))

    xlin, aa = _project(x, W, a2)
    a_src = aa[:, 0]
    a_dst = aa[:, 1]
    src = edge_index[0]
    dst = edge_index[1]

    acc = _sc_edges(a_src, a_dst, src, dst, xlin)
    return _combine(acc, xlin, aa)


# speedup vs baseline: 23.2165x; 23.2165x over previous
import functools

import jax
import jax.numpy as jnp
from jax import lax
from jax.experimental import pallas as pl
from jax.experimental.pallas import tpu as pltpu
from jax.experimental.pallas import tpu_sc as plsc

N = 10000
E = 320000
D = 128
DH = D // 2
NEG_SLOPE = 0.2

NC = 2
NS = 16
NW = NC * NS
C = 128
NCHUNK = E // C
ROWS_PER_TILE = 624
TAIL_ROWS = N - NS * ROWS_PER_TILE


def _project_body(x_ref, w_ref, a2_ref, xlin_ref, aa_ref):
    xl = jnp.dot(x_ref[...], w_ref[...], preferred_element_type=jnp.float32)
    xlin_ref[...] = xl
    aa_ref[...] = jnp.dot(xl, a2_ref[...], preferred_element_type=jnp.float32)


def _project(x, W, a2):
    return pl.pallas_call(
        _project_body,
        out_shape=(
            jax.ShapeDtypeStruct((N, D), jnp.float32),
            jax.ShapeDtypeStruct((N, D), jnp.float32),
        ),
    )(x, W, a2)


def _sc_body(a_src_hbm, a_dst_hbm, edge_hbm, xlin2_hbm,
             msg_hbm, den_hbm,
             a_src_v, a_dst_v, den_v, src_v, dst_v, src2_v, rows_v, msg_v,
             w_v, acc_sh, sem):
    cid = lax.axis_index("c")
    sid = lax.axis_index("s")
    wid = sid * NC + cid

    pltpu.sync_copy(a_src_hbm, a_src_v)
    pltpu.sync_copy(a_dst_hbm, a_dst_v)

    zeros16 = jnp.zeros((16,), jnp.float32)

    def zero_den(r, _):
        den_v[pl.ds(r * 16, 16)] = zeros16
        return 0

    lax.fori_loop(0, N // 16, zero_den, 0)

    base_row = sid * ROWS_PER_TILE
    nk = (NCHUNK - wid + NW - 1) // NW

    def phase(p, _):
        def zero_row(r, _):
            for c16 in range(DH // 16):
                msg_v[r, pl.ds(c16 * 16, 16)] = zeros16
            return 0

        lax.fori_loop(0, C, zero_row, 0)

        for i in range(4):
            pltpu.sync_copy(msg_v, acc_sh.at[pl.ds(base_row + i * C, C)])
        rem = ROWS_PER_TILE - 4 * C
        pltpu.sync_copy(msg_v.at[pl.ds(0, rem)],
                        acc_sh.at[pl.ds(base_row + 4 * C, rem)])

        @pl.when(sid == NS - 1)
        def _zero_tail():
            pltpu.sync_copy(msg_v.at[pl.ds(0, TAIL_ROWS)],
                            acc_sh.at[pl.ds(NS * ROWS_PER_TILE, TAIL_ROWS)])

        plsc.subcore_barrier()

        def chunk(i, _):
            k = wid + i * NW
            base = k * C
            pltpu.sync_copy(edge_hbm.at[0, pl.ds(base, C)], src_v)
            pltpu.sync_copy(edge_hbm.at[1, pl.ds(base, C)], dst_v)

            def grp(g, _):
                sl = pl.ds(g * 16, 16)
                sidx = src_v[sl]
                didx = dst_v[sl]
                logit = (plsc.load_gather(a_src_v, [sidx])
                         + plsc.load_gather(a_dst_v, [didx]))
                w16 = jnp.exp(jnp.where(logit >= 0, logit,
                                        NEG_SLOPE * logit))
                w_v[sl] = w16
                src2_v[sl] = sidx * 2 + p
                plsc.addupdate_scatter(
                    den_v, [didx], w16,
                    mask=jnp.full((16,), p == 0, jnp.bool_))
                return 0

            lax.fori_loop(0, C // 16, grp, 0)

            pltpu.async_copy(xlin2_hbm.at[src2_v], rows_v, sem).wait()

            def edge16(g, _):
                w16 = w_v[pl.ds(g * 16, 16)]
                for j in range(16):
                    e = g * 16 + j
                    wv = jnp.full((16,), w16[j], jnp.float32)
                    for c16 in range(DH // 16):
                        sl = pl.ds(c16 * 16, 16)
                        msg_v[e, sl] = rows_v[e, sl] * wv
                return 0

            lax.fori_loop(0, C // 16, edge16, 0)

            pltpu.sync_copy(msg_v, acc_sh.at[dst_v], add=True)
            return 0

        lax.fori_loop(0, nk, chunk, 0)
        plsc.subcore_barrier()

        pltpu.sync_copy(acc_sh.at[pl.ds(base_row, ROWS_PER_TILE)],
                        msg_hbm.at[cid, p, pl.ds(base_row, ROWS_PER_TILE)])

        @pl.when(sid == NS - 1)
        def _flush_tail():
            pltpu.sync_copy(
                acc_sh.at[pl.ds(NS * ROWS_PER_TILE, TAIL_ROWS)],
                msg_hbm.at[cid, p, pl.ds(NS * ROWS_PER_TILE, TAIL_ROWS)])

        plsc.subcore_barrier()
        return 0

    lax.fori_loop(0, 2, phase, 0)

    pltpu.sync_copy(den_v, den_hbm.at[wid])


_sc_edges = functools.partial(
    pl.kernel,
    out_type=(
        jax.ShapeDtypeStruct((NC, 2, N, DH), jnp.float32),
        jax.ShapeDtypeStruct((NW, N), jnp.float32),
    ),
    mesh=plsc.VectorSubcoreMesh(core_axis_name="c", subcore_axis_name="s"),
    compiler_params=pltpu.CompilerParams(
        needs_layout_passes=False, use_tc_tiling_on_sc=False),
    scratch_types=[
        pltpu.VMEM((N,), jnp.float32),
        pltpu.VMEM((N,), jnp.float32),
        pltpu.VMEM((N,), jnp.float32),
        pltpu.VMEM((C,), jnp.int32),
        pltpu.VMEM((C,), jnp.int32),
        pltpu.VMEM((C,), jnp.int32),
        pltpu.VMEM((C, DH), jnp.float32),
        pltpu.VMEM((C, DH), jnp.float32),
        pltpu.VMEM((C + 16,), jnp.float32),
        pltpu.VMEM_SHARED((N, DH), jnp.float32),
        pltpu.SemaphoreType.DMA,
    ],
)(_sc_body)


def _combine_body(msg_ref, den_ref, xlin_ref, aa_ref, out_ref):
    den = jnp.sum(den_ref[...], axis=0)[:, None]
    asum = aa_ref[:, 0:1] + aa_ref[:, 1:2]
    wself = jnp.exp(jnp.where(asum >= 0, asum, NEG_SLOPE * asum))
    inv = 1.0 / (den + wself + 1e-16)
    msgl = msg_ref[0, 0] + msg_ref[1, 0]
    msgr = msg_ref[0, 1] + msg_ref[1, 1]
    out_ref[:, :DH] = (msgl + wself * xlin_ref[:, :DH]) * inv
    out_ref[:, DH:] = (msgr + wself * xlin_ref[:, DH:]) * inv


def _combine(msg, den, xlin, aa):
    return pl.pallas_call(
        _combine_body,
        out_shape=jax.ShapeDtypeStruct((N, D), jnp.float32),
    )(msg, den, xlin, aa)


def kernel(x, edge_index, W, att_src, att_dst):
    a2 = jnp.zeros((D, D), jnp.float32)
    a2 = a2.at[:, 0].set(att_src.reshape(-1)).at[:, 1].set(att_dst.reshape(-1))

    xlin, aa = _project(x, W, a2)
    a_src = aa[:, 0]
    a_dst = aa[:, 1]
    xlin2 = xlin.reshape(2 * N, DH)

    msg, den = _sc_edges(a_src, a_dst, edge_index, xlin2)
    return _combine(msg, den, xlin, aa)


# speedup vs baseline: 39.5361x; 1.7029x over previous
import functools

import jax
import jax.numpy as jnp
from jax import lax
from jax.experimental import pallas as pl
from jax.experimental.pallas import tpu as pltpu
from jax.experimental.pallas import tpu_sc as plsc

N = 10000
E = 320000
D = 128
DH = D // 2
NEG_SLOPE = 0.2

NC = 2
NS = 16
NW = NC * NS
C = 128
NCHUNK = E // C
ROWS_PER_TILE = 624
TAIL_ROWS = N - NS * ROWS_PER_TILE


def _project_body(x_ref, w_ref, a2_ref, xlin_ref, aa_ref):
    xl = jnp.dot(x_ref[...], w_ref[...], preferred_element_type=jnp.float32)
    xlin_ref[...] = xl
    aa_ref[...] = jnp.dot(xl, a2_ref[...], preferred_element_type=jnp.float32)


def _project(x, W, a2):
    return pl.pallas_call(
        _project_body,
        out_shape=(
            jax.ShapeDtypeStruct((N, D), jnp.float32),
            jax.ShapeDtypeStruct((N, D), jnp.float32),
        ),
    )(x, W, a2)


def _sc_body(a_src_hbm, a_dst_hbm, edge_hbm, xlin2_hbm,
             msg_hbm, den_hbm,
             a_src_v, a_dst_v, den_v,
             src_v0, src_v1, dst_v0, dst_v1, idx2_v0, idx2_v1,
             rows_v0, rows_v1, w_v0, w_v1, msg_v0, msg_v1, acc_sh,
             semi0, semi1, semg0, semg1, sems0, sems1):
    cid = lax.axis_index("c")
    sid = lax.axis_index("s")
    wid = sid * NC + cid

    srcs = (src_v0, src_v1)
    dsts = (dst_v0, dst_v1)
    idx2s = (idx2_v0, idx2_v1)
    rows = (rows_v0, rows_v1)
    ws = (w_v0, w_v1)
    msgs = (msg_v0, msg_v1)
    semI = (semi0, semi1)
    semG = (semg0, semg1)
    semS = (sems0, sems1)

    pltpu.sync_copy(a_src_hbm, a_src_v)
    pltpu.sync_copy(a_dst_hbm, a_dst_v)

    zeros16 = jnp.zeros((16,), jnp.float32)

    def zero_den(r, _):
        den_v[pl.ds(r * 16, 16)] = zeros16
        return 0

    lax.fori_loop(0, N // 16, zero_den, 0)

    base_row = sid * ROWS_PER_TILE
    nk = (NCHUNK - wid + NW - 1) // NW

    def start_idx(i, b):
        base = (wid + i * NW) * C
        pltpu.async_copy(edge_hbm.at[0, pl.ds(base, C)], srcs[b], semI[b])
        pltpu.async_copy(edge_hbm.at[1, pl.ds(base, C)], dsts[b], semI[b])

    def phase(p, _):
        def zero_row(r, _):
            for c16 in range(DH // 16):
                msg_v0[r, pl.ds(c16 * 16, 16)] = zeros16
            return 0

        lax.fori_loop(0, C, zero_row, 0)

        for i in range(4):
            pltpu.sync_copy(msg_v0, acc_sh.at[pl.ds(base_row + i * C, C)])
        rem = ROWS_PER_TILE - 4 * C
        pltpu.sync_copy(msg_v0.at[pl.ds(0, rem)],
                        acc_sh.at[pl.ds(base_row + 4 * C, rem)])

        @pl.when(sid == NS - 1)
        def _zero_tail():
            pltpu.sync_copy(msg_v0.at[pl.ds(0, TAIL_ROWS)],
                            acc_sh.at[pl.ds(NS * ROWS_PER_TILE, TAIL_ROWS)])

        plsc.subcore_barrier()

        start_idx(0, 0)

        def pair(ii, _):
            for b in range(2):
                o = 1 - b
                i = 2 * ii + b

                @pl.when(i < nk)
                def _chunk():
                    pltpu.make_async_copy(edge_hbm.at[0, pl.ds(0, C)],
                                          srcs[b], semI[b]).wait()
                    pltpu.make_async_copy(edge_hbm.at[1, pl.ds(0, C)],
                                          dsts[b], semI[b]).wait()

                    def mini(g, _):
                        sl = pl.ds(g * 16, 16)
                        idx2s[b][sl] = srcs[b][sl] * 2 + p
                        return 0

                    lax.fori_loop(0, C // 16, mini, 0)
                    gcp = pltpu.async_copy(xlin2_hbm.at[idx2s[b]],
                                           rows[b], semG[b])

                    @pl.when(i >= 1)
                    def _drain():
                        pltpu.make_async_copy(
                            msgs[o], acc_sh.at[dsts[o]], semS[o]).wait()

                    @pl.when(i + 1 < nk)
                    def _prefetch():
                        start_idx(i + 1, o)

                    def grp(g, _):
                        sl = pl.ds(g * 16, 16)
                        sidx = srcs[b][sl]
                        didx = dsts[b][sl]
                        logit = (plsc.load_gather(a_src_v, [sidx])
                                 + plsc.load_gather(a_dst_v, [didx]))
                        w16 = jnp.exp(jnp.where(logit >= 0, logit,
                                                NEG_SLOPE * logit))
                        ws[b][sl] = w16
                        plsc.addupdate_scatter(
                            den_v, [didx], w16,
                            mask=jnp.full((16,), p == 0, jnp.bool_))
                        return 0

                    lax.fori_loop(0, C // 16, grp, 0)

                    gcp.wait()

                    def edge16(g, _):
                        w16 = ws[b][pl.ds(g * 16, 16)]
                        for j in range(16):
                            e = g * 16 + j
                            wv = jnp.full((16,), w16[j], jnp.float32)
                            for c16 in range(DH // 16):
                                sl = pl.ds(c16 * 16, 16)
                                msgs[b][e, sl] = rows[b][e, sl] * wv
                        return 0

                    lax.fori_loop(0, C // 16, edge16, 0)

                    pltpu.async_copy(msgs[b], acc_sh.at[dsts[b]],
                                     semS[b], add=True)

            return 0

        lax.fori_loop(0, (nk + 1) // 2, pair, 0)

        @pl.when(nk % 2 == 1)
        def _d0():
            pltpu.make_async_copy(msgs[0], acc_sh.at[dsts[0]],
                                  semS[0]).wait()

        @pl.when(nk % 2 == 0)
        def _d1():
            pltpu.make_async_copy(msgs[1], acc_sh.at[dsts[1]],
                                  semS[1]).wait()

        plsc.subcore_barrier()

        pltpu.sync_copy(acc_sh.at[pl.ds(base_row, ROWS_PER_TILE)],
                        msg_hbm.at[cid, p, pl.ds(base_row, ROWS_PER_TILE)])

        @pl.when(sid == NS - 1)
        def _flush_tail():
            pltpu.sync_copy(
                acc_sh.at[pl.ds(NS * ROWS_PER_TILE, TAIL_ROWS)],
                msg_hbm.at[cid, p, pl.ds(NS * ROWS_PER_TILE, TAIL_ROWS)])

        plsc.subcore_barrier()
        return 0

    lax.fori_loop(0, 2, phase, 0)

    pltpu.sync_copy(den_v, den_hbm.at[wid])


_sc_edges = functools.partial(
    pl.kernel,
    out_type=(
        jax.ShapeDtypeStruct((NC, 2, N, DH), jnp.float32),
        jax.ShapeDtypeStruct((NW, N), jnp.float32),
    ),
    mesh=plsc.VectorSubcoreMesh(core_axis_name="c", subcore_axis_name="s"),
    compiler_params=pltpu.CompilerParams(
        needs_layout_passes=False, use_tc_tiling_on_sc=False),
    scratch_types=[
        pltpu.VMEM((N,), jnp.float32),
        pltpu.VMEM((N,), jnp.float32),
        pltpu.VMEM((N,), jnp.float32),
        pltpu.VMEM((C,), jnp.int32),
        pltpu.VMEM((C,), jnp.int32),
        pltpu.VMEM((C,), jnp.int32),
        pltpu.VMEM((C,), jnp.int32),
        pltpu.VMEM((C,), jnp.int32),
        pltpu.VMEM((C,), jnp.int32),
        pltpu.VMEM((C, DH), jnp.float32),
        pltpu.VMEM((C, DH), jnp.float32),
        pltpu.VMEM((C,), jnp.float32),
        pltpu.VMEM((C,), jnp.float32),
        pltpu.VMEM((C, DH), jnp.float32),
        pltpu.VMEM((C, DH), jnp.float32),
        pltpu.VMEM_SHARED((N, DH), jnp.float32),
        pltpu.SemaphoreType.DMA,
        pltpu.SemaphoreType.DMA,
        pltpu.SemaphoreType.DMA,
        pltpu.SemaphoreType.DMA,
        pltpu.SemaphoreType.DMA,
        pltpu.SemaphoreType.DMA,
    ],
)(_sc_body)


def _combine_body(msg_ref, den_ref, xlin_ref, aa_ref, out_ref):
    den = jnp.sum(den_ref[...], axis=0)[:, None]
    asum = aa_ref[:, 0:1] + aa_ref[:, 1:2]
    wself = jnp.exp(jnp.where(asum >= 0, asum, NEG_SLOPE * asum))
    inv = 1.0 / (den + wself + 1e-16)
    msgl = msg_ref[0, 0] + msg_ref[1, 0]
    msgr = msg_ref[0, 1] + msg_ref[1, 1]
    out_ref[:, :DH] = (msgl + wself * xlin_ref[:, :DH]) * inv
    out_ref[:, DH:] = (msgr + wself * xlin_ref[:, DH:]) * inv


def _combine(msg, den, xlin, aa):
    return pl.pallas_call(
        _combine_body,
        out_shape=jax.ShapeDtypeStruct((N, D), jnp.float32),
    )(msg, den, xlin, aa)


def kernel(x, edge_index, W, att_src, att_dst):
    a2 = jnp.zeros((D, D), jnp.float32)
    a2 = a2.at[:, 0].set(att_src.reshape(-1)).at[:, 1].set(att_dst.reshape(-1))

    xlin, aa = _project(x, W, a2)
    a_src = aa[:, 0]
    a_dst = aa[:, 1]
    xlin2 = xlin.reshape(2 * N, DH)

    msg, den = _sc_edges(a_src, a_dst, edge_index, xlin2)
    return _combine(msg, den, xlin, aa)


# speedup vs baseline: 40.1711x; 1.0161x over previous
import functools

import jax
import jax.numpy as jnp
from jax import lax
from jax.experimental import pallas as pl
from jax.experimental.pallas import tpu as pltpu
from jax.experimental.pallas import tpu_sc as plsc

N = 10000
E = 320000
D = 128
DH = D // 2
NEG_SLOPE = 0.2

NC = 2
NS = 16
NW = NC * NS
C = 128
NCHUNK = E // C
ROWS_PER_TILE = 624
TAIL_ROWS = N - NS * ROWS_PER_TILE


def _project_body(x_ref, w_ref, a2_ref, xlin_ref, aa_ref):
    xl = jnp.dot(x_ref[...], w_ref[...], preferred_element_type=jnp.float32)
    xlin_ref[...] = xl
    aa_ref[...] = jnp.dot(xl, a2_ref[...], preferred_element_type=jnp.float32)


def _project(x, W, a2):
    return pl.pallas_call(
        _project_body,
        out_shape=(
            jax.ShapeDtypeStruct((N, D), jnp.float32),
            jax.ShapeDtypeStruct((N, D), jnp.float32),
        ),
    )(x, W, a2)


def _sc_body(a_src_hbm, a_dst_hbm, edge_hbm, xlin2_hbm,
             msg_hbm, den_hbm,
             a_src_v, a_dst_v, den_v,
             src_v0, src_v1, dst_v0, dst_v1, idx2_v0, idx2_v1,
             rows_v0, rows_v1, w_v0, w_v1, msg_v0, msg_v1, acc_sh,
             semi0, semi1, semg0, semg1, sems0, sems1):
    cid = lax.axis_index("c")
    sid = lax.axis_index("s")
    wid = sid * NC + cid

    srcs = (src_v0, src_v1)
    dsts = (dst_v0, dst_v1)
    idx2s = (idx2_v0, idx2_v1)
    rows = (rows_v0, rows_v1)
    ws = (w_v0, w_v1)
    msgs = (msg_v0, msg_v1)
    semI = (semi0, semi1)
    semG = (semg0, semg1)
    semS = (sems0, sems1)

    pltpu.sync_copy(a_src_hbm, a_src_v)
    pltpu.sync_copy(a_dst_hbm, a_dst_v)

    zeros16 = jnp.zeros((16,), jnp.float32)

    def zero_den(r, _):
        den_v[pl.ds(r * 16, 16)] = zeros16
        return 0

    lax.fori_loop(0, N // 16, zero_den, 0)

    base_row = sid * ROWS_PER_TILE
    nk = (NCHUNK - wid + NW - 1) // NW

    def start_idx(i, b):
        base = (wid + i * NW) * C
        pltpu.async_copy(edge_hbm.at[0, pl.ds(base, C)], srcs[b], semI[b])
        pltpu.async_copy(edge_hbm.at[1, pl.ds(base, C)], dsts[b], semI[b])

    def phase(p, _):
        def zero_row(r, _):
            for c16 in range(DH // 16):
                msg_v0[r, pl.ds(c16 * 16, 16)] = zeros16
            return 0

        lax.fori_loop(0, C, zero_row, 0)

        for i in range(4):
            pltpu.sync_copy(msg_v0, acc_sh.at[pl.ds(base_row + i * C, C)])
        rem = ROWS_PER_TILE - 4 * C
        pltpu.sync_copy(msg_v0.at[pl.ds(0, rem)],
                        acc_sh.at[pl.ds(base_row + 4 * C, rem)])

        @pl.when(sid == NS - 1)
        def _zero_tail():
            pltpu.sync_copy(msg_v0.at[pl.ds(0, TAIL_ROWS)],
                            acc_sh.at[pl.ds(NS * ROWS_PER_TILE, TAIL_ROWS)])

        plsc.subcore_barrier()

        start_idx(0, 0)

        def pair(ii, _):
            for b in range(2):
                o = 1 - b
                i = 2 * ii + b

                @pl.when(i < nk)
                def _chunk():
                    pltpu.make_async_copy(edge_hbm.at[0, pl.ds(0, C)],
                                          srcs[b], semI[b]).wait()
                    pltpu.make_async_copy(edge_hbm.at[1, pl.ds(0, C)],
                                          dsts[b], semI[b]).wait()

                    def mini(g, _):
                        sl = pl.ds(g * 16, 16)
                        idx2s[b][sl] = srcs[b][sl] * 2 + p
                        return 0

                    lax.fori_loop(0, C // 16, mini, 0)
                    gcp = pltpu.async_copy(xlin2_hbm.at[idx2s[b]],
                                           rows[b], semG[b])

                    @pl.when(i >= 1)
                    def _drain():
                        pltpu.make_async_copy(
                            msgs[o], acc_sh.at[dsts[o]], semS[o]).wait()

                    @pl.when(i + 1 < nk)
                    def _prefetch():
                        start_idx(i + 1, o)

                    def grp(g, _):
                        sl = pl.ds(g * 16, 16)
                        sidx = srcs[b][sl]
                        didx = dsts[b][sl]
                        logit = (plsc.load_gather(a_src_v, [sidx])
                                 + plsc.load_gather(a_dst_v, [didx]))
                        w16 = jnp.exp(jnp.where(logit >= 0, logit,
                                                NEG_SLOPE * logit))
                        ws[b][sl] = w16
                        plsc.addupdate_scatter(
                            den_v, [didx], w16,
                            mask=jnp.full((16,), p == 0, jnp.bool_))
                        return 0

                    lax.fori_loop(0, C // 16, grp, 0)

                    gcp.wait()

                    @plsc.parallel_loop(0, C // 16, unroll=2)
                    def edge16(g):
                        w16 = ws[b][pl.ds(g * 16, 16)]
                        for j in range(16):
                            e = g * 16 + j
                            wv = jnp.full((16,), w16[j], jnp.float32)
                            for c16 in range(DH // 16):
                                sl = pl.ds(c16 * 16, 16)
                                msgs[b][e, sl] = rows[b][e, sl] * wv

                    pltpu.async_copy(msgs[b], acc_sh.at[dsts[b]],
                                     semS[b], add=True)

            return 0

        lax.fori_loop(0, (nk + 1) // 2, pair, 0)

        @pl.when(nk % 2 == 1)
        def _d0():
            pltpu.make_async_copy(msgs[0], acc_sh.at[dsts[0]],
                                  semS[0]).wait()

        @pl.when(nk % 2 == 0)
        def _d1():
            pltpu.make_async_copy(msgs[1], acc_sh.at[dsts[1]],
                                  semS[1]).wait()

        plsc.subcore_barrier()

        pltpu.sync_copy(acc_sh.at[pl.ds(base_row, ROWS_PER_TILE)],
                        msg_hbm.at[cid, p, pl.ds(base_row, ROWS_PER_TILE)])

        @pl.when(sid == NS - 1)
        def _flush_tail():
            pltpu.sync_copy(
                acc_sh.at[pl.ds(NS * ROWS_PER_TILE, TAIL_ROWS)],
                msg_hbm.at[cid, p, pl.ds(NS * ROWS_PER_TILE, TAIL_ROWS)])

        plsc.subcore_barrier()
        return 0

    lax.fori_loop(0, 2, phase, 0)

    pltpu.sync_copy(den_v, den_hbm.at[wid])


_sc_edges = functools.partial(
    pl.kernel,
    out_type=(
        jax.ShapeDtypeStruct((NC, 2, N, DH), jnp.float32),
        jax.ShapeDtypeStruct((NW, N), jnp.float32),
    ),
    mesh=plsc.VectorSubcoreMesh(core_axis_name="c", subcore_axis_name="s"),
    compiler_params=pltpu.CompilerParams(
        needs_layout_passes=False, use_tc_tiling_on_sc=False),
    scratch_types=[
        pltpu.VMEM((N,), jnp.float32),
        pltpu.VMEM((N,), jnp.float32),
        pltpu.VMEM((N,), jnp.float32),
        pltpu.VMEM((C,), jnp.int32),
        pltpu.VMEM((C,), jnp.int32),
        pltpu.VMEM((C,), jnp.int32),
        pltpu.VMEM((C,), jnp.int32),
        pltpu.VMEM((C,), jnp.int32),
        pltpu.VMEM((C,), jnp.int32),
        pltpu.VMEM((C, DH), jnp.float32),
        pltpu.VMEM((C, DH), jnp.float32),
        pltpu.VMEM((C,), jnp.float32),
        pltpu.VMEM((C,), jnp.float32),
        pltpu.VMEM((C, DH), jnp.float32),
        pltpu.VMEM((C, DH), jnp.float32),
        pltpu.VMEM_SHARED((N, DH), jnp.float32),
        pltpu.SemaphoreType.DMA,
        pltpu.SemaphoreType.DMA,
        pltpu.SemaphoreType.DMA,
        pltpu.SemaphoreType.DMA,
        pltpu.SemaphoreType.DMA,
        pltpu.SemaphoreType.DMA,
    ],
)(_sc_body)


def _combine_body(msg_ref, den_ref, xlin_ref, aa_ref, out_ref):
    den = jnp.sum(den_ref[...], axis=0)[:, None]
    asum = aa_ref[:, 0:1] + aa_ref[:, 1:2]
    wself = jnp.exp(jnp.where(asum >= 0, asum, NEG_SLOPE * asum))
    inv = 1.0 / (den + wself + 1e-16)
    msgl = msg_ref[0, 0] + msg_ref[1, 0]
    msgr = msg_ref[0, 1] + msg_ref[1, 1]
    out_ref[:, :DH] = (msgl + wself * xlin_ref[:, :DH]) * inv
    out_ref[:, DH:] = (msgr + wself * xlin_ref[:, DH:]) * inv


def _combine(msg, den, xlin, aa):
    return pl.pallas_call(
        _combine_body,
        out_shape=jax.ShapeDtypeStruct((N, D), jnp.float32),
    )(msg, den, xlin, aa)


def kernel(x, edge_index, W, att_src, att_dst):
    a2 = jnp.zeros((D, D), jnp.float32)
    a2 = a2.at[:, 0].set(att_src.reshape(-1)).at[:, 1].set(att_dst.reshape(-1))

    xlin, aa = _project(x, W, a2)
    a_src = aa[:, 0]
    a_dst = aa[:, 1]
    xlin2 = xlin.reshape(2 * N, DH)

    msg, den = _sc_edges(a_src, a_dst, edge_index, xlin2)
    return _combine(msg, den, xlin, aa)
